# Initial kernel scaffold; baseline (speedup 1.0000x reference)
import jax
import jax.numpy as jnp
from jax.experimental import pallas as pl


def kernel(x, tables, Wc, bc, W, b):
    raise NotImplementedError("write your pallas kernel here")


# speedup vs baseline: 2.0969x; 2.0969x over previous
import functools

import jax
import jax.numpy as jnp
from jax import lax
from jax.experimental import pallas as pl
from jax.experimental.pallas import tpu as pltpu
from jax.experimental.pallas import tpu_sc as plsc

B = 16384
NF = 21
V = 100000
EMB = 16
XW = 22

NC = 2
NS = 16
NW = NC * NS
BPW = B // NW
RPW = BPW * NF
CH = 128
NCH = RPW // CH
NBUF = 4
LANES = 16


def _sc_body(x_hbm, tbl_hbm, out_hbm, xbuf, idxbuf, rows, s0, s1, s2, s3):
    sems = (s0, s1, s2, s3)
    wid = lax.axis_index("s") * NC + lax.axis_index("c")
    base = wid * BPW

    pltpu.sync_copy(x_hbm.at[pl.ds(base * XW, BPW * XW)], xbuf)

    @pl.loop(0, RPW // LANES)
    def _(j):
        p = j * LANES + lax.iota(jnp.int32, LANES)
        bb = p // NF
        src = p + bb
        v = plsc.load_gather(xbuf, [src])
        f = p - bb * NF
        g = v + f * V
        row = j // (CH // LANES)
        col = (j % (CH // LANES)) * LANES
        idxbuf[row, pl.ds(col, LANES)] = g

    def start(c, t):
        pltpu.async_copy(tbl_hbm.at[idxbuf.at[c]], rows.at[t], sems[t])

    def wait(c, t):
        pltpu.make_async_copy(tbl_hbm.at[idxbuf.at[c]], rows.at[t],
                              sems[t]).wait()

    def write_out(c, t):
        pltpu.sync_copy(rows.at[t], out_hbm.at[pl.ds(wid * RPW + c * CH, CH)])

    for t in range(NBUF):
        start(t, t)

    @pl.loop(0, NCH - NBUF, step=NBUF)
    def _(c0):
        for t in range(NBUF):
            c = c0 + t
            wait(c, t)
            write_out(c, t)
            start(c + NBUF, t)

    for t in range(NBUF):
        c = NCH - NBUF + t
        wait(c, t)
        write_out(c, t)


_sc_gather = functools.partial(
    pl.kernel,
    out_type=jax.ShapeDtypeStruct((B * NF, EMB), jnp.float32),
    mesh=plsc.VectorSubcoreMesh(core_axis_name="c", subcore_axis_name="s",
                                num_cores=NC, num_subcores=NS),
    scratch_types=[
        pltpu.VMEM((BPW * XW,), jnp.int32),
        pltpu.VMEM((NCH, CH), jnp.int32),
        pltpu.VMEM((NBUF, CH, EMB), jnp.float32),
        pltpu.SemaphoreType.DMA,
        pltpu.SemaphoreType.DMA,
        pltpu.SemaphoreType.DMA,
        pltpu.SemaphoreType.DMA,
    ],
    compiler_params=pltpu.CompilerParams(use_tc_tiling_on_sc=False,
                                         needs_layout_passes=False),
)(_sc_body)


BBLK = 2048


def _tc_body(g_ref, x_ref, wc_ref, bc_ref, w_ref, b_ref, o_ref):
    cont = x_ref[:, NF:NF + 1].astype(jnp.float32)
    pc = cont * wc_ref[0:1, :] + bc_ref[0:1, :]
    pc = jnp.where(pc >= 0, pc, 0.01 * pc)
    acc = jnp.dot(g_ref[...], w_ref[0:NF * EMB, :],
                  preferred_element_type=jnp.float32)
    acc = acc + jnp.dot(pc, w_ref[NF * EMB:, :],
                        preferred_element_type=jnp.float32)
    acc = acc + b_ref[0:1, :]
    o_ref[...] = jnp.where(acc >= 0, acc, 0.01 * acc)


def _tc_head(gmat, x, wc, bc, w, b):
    grid = (B // BBLK,)
    return pl.pallas_call(
        _tc_body,
        grid=grid,
        in_specs=[
            pl.BlockSpec((BBLK, NF * EMB), lambda j: (j, 0)),
            pl.BlockSpec((BBLK, XW), lambda j: (j, 0)),
            pl.BlockSpec((1, 8), lambda j: (0, 0)),
            pl.BlockSpec((1, 8), lambda j: (0, 0)),
            pl.BlockSpec((NF * EMB + 8, 32), lambda j: (0, 0)),
            pl.BlockSpec((1, 32), lambda j: (0, 0)),
        ],
        out_specs=pl.BlockSpec((BBLK, 32), lambda j: (j, 0)),
        out_shape=jax.ShapeDtypeStruct((B, 32), jnp.float32),
    )(gmat, x, wc, bc, w, b)


def kernel(x, tables, Wc, bc, W, b):
    x_flat = x.reshape(-1)
    tbl_flat = tables.reshape(NF * V, EMB)
    g = _sc_gather(x_flat, tbl_flat)
    gmat = g.reshape(B, NF * EMB)
    return _tc_head(gmat, x, Wc.reshape(1, 8), bc.reshape(1, 8), W,
                    b.reshape(1, 32))


# speedup vs baseline: 18.0360x; 8.6014x over previous
import functools

import jax
import jax.numpy as jnp
from jax import lax
from jax.experimental import pallas as pl
from jax.experimental.pallas import tpu as pltpu
from jax.experimental.pallas import tpu_sc as plsc

B = 16384
NF = 21
V = 100000
EMB = 16
XW = 22
COLS = NF * EMB

NC = 2
NS = 16
NW = NC * NS
LANES = 16
QB = B // 4


def _sc_body(xt_hbm, tt_hbm, gt_hbm, idxb, tvec, gbuf, semw0, semw1):
    semw = (semw0, semw1)
    wid = lax.axis_index("s") * NC + lax.axis_index("c")
    sw = 10 * wid + jnp.minimum(wid, 16)
    ncol = jnp.where(wid < 16, 11, 10)

    @pl.loop(0, ncol, init_carry=jnp.int32(-1))
    def _(k, fprev):
        col = sw + k
        f = col // EMB
        e = col - f * EMB

        @pl.when(f != fprev)
        def _():
            pltpu.sync_copy(xt_hbm.at[f], idxb)

        pltpu.sync_copy(tt_hbm.at[f, e], tvec)
        for q in range(4):
            t = q % 2
            if q >= 2:
                pltpu.make_async_copy(
                    gbuf.at[t], gt_hbm.at[col, pl.ds((q - 2) * QB, QB)],
                    semw[t]).wait()

            @plsc.parallel_loop(0, QB // LANES, unroll=16)
            def _(kk):
                iv = idxb[pl.ds(q * QB + kk * LANES, LANES)]
                gbuf[t, pl.ds(kk * LANES, LANES)] = plsc.load_gather(
                    tvec, [iv])
            pltpu.async_copy(gbuf.at[t], gt_hbm.at[col, pl.ds(q * QB, QB)],
                             semw[t])
        for q in (2, 3):
            pltpu.make_async_copy(
                gbuf.at[q % 2], gt_hbm.at[col, pl.ds(q * QB, QB)],
                semw[q % 2]).wait()
        return f


_sc_gather = functools.partial(
    pl.kernel,
    out_type=jax.ShapeDtypeStruct((COLS, B), jnp.float32),
    mesh=plsc.VectorSubcoreMesh(core_axis_name="c", subcore_axis_name="s",
                                num_cores=NC, num_subcores=NS),
    scratch_types=[
        pltpu.VMEM((B,), jnp.int32),
        pltpu.VMEM((V,), jnp.float32),
        pltpu.VMEM((2, QB), jnp.float32),
        pltpu.SemaphoreType.DMA,
        pltpu.SemaphoreType.DMA,
    ],
    compiler_params=pltpu.CompilerParams(use_tc_tiling_on_sc=True,
                                         needs_layout_passes=False),
)(_sc_body)


BBLK = 4096


def _tc_body(g_ref, x_ref, wc_ref, bc_ref, w_ref, b_ref, o_ref):
    cont = x_ref[NF:NF + 1, :].astype(jnp.float32)
    pc = wc_ref[...] * cont + bc_ref[...]
    pc = jnp.where(pc >= 0, pc, 0.01 * pc)
    acc = jnp.dot(w_ref[:, 0:COLS], g_ref[...],
                  preferred_element_type=jnp.float32)
    acc = acc + jnp.dot(w_ref[:, COLS:], pc,
                        preferred_element_type=jnp.float32)
    acc = acc + b_ref[...]
    o_ref[...] = jnp.where(acc >= 0, acc, 0.01 * acc)


def _tc_head(gt, xt, wct, bc, wt, b):
    grid = (B // BBLK,)
    return pl.pallas_call(
        _tc_body,
        grid=grid,
        in_specs=[
            pl.BlockSpec((COLS, BBLK), lambda j: (0, j)),
            pl.BlockSpec((XW, BBLK), lambda j: (0, j)),
            pl.BlockSpec((8, 1), lambda j: (0, 0)),
            pl.BlockSpec((8, 1), lambda j: (0, 0)),
            pl.BlockSpec((32, COLS + 8), lambda j: (0, 0)),
            pl.BlockSpec((32, 1), lambda j: (0, 0)),
        ],
        out_specs=pl.BlockSpec((32, BBLK), lambda j: (0, j)),
        out_shape=jax.ShapeDtypeStruct((32, B), jnp.float32),
    )(gt, xt, wct, bc, wt, b)


def kernel(x, tables, Wc, bc, W, b):
    xt = x.T
    tt = jnp.transpose(tables, (0, 2, 1))
    gt = _sc_gather(xt, tt)
    outt = _tc_head(gt, xt, Wc.T, bc.reshape(8, 1), W.T, b.reshape(32, 1))
    return outt.T
